# baseline (device time: 900748 ns/iter reference)
import jax
import jax.numpy as jnp
from jax import lax
from jax.experimental import pallas as pl
from jax.experimental.pallas import tpu as pltpu

T = 1024
V = 32768
V_HALF = V // 2
C = 32
BM = T // C


def _make_chunk_rdma(l_any_ref, lrem_ref, send_sems, recv_sems, nbr, j):
    return pltpu.make_async_remote_copy(
        src_ref=l_any_ref.at[pl.ds(j * BM, BM), :],
        dst_ref=lrem_ref.at[pl.ds(j * BM, BM), :],
        send_sem=send_sems.at[j],
        recv_sem=recv_sems.at[j],
        device_id=nbr,
        device_id_type=pl.DeviceIdType.MESH,
    )


def _fused_body(
    l_any_ref,
    la_ref,
    out_ref,
    lrem_ref,
    lb_vmem,
    send_sems,
    recv_sems,
    copy_sem,
):
    step = pl.program_id(0)
    my_x = lax.axis_index("x")
    my_y = lax.axis_index("y")
    my_z = lax.axis_index("z")
    nbr = (1 - my_x, my_y, my_z)

    @pl.when(step == 0)
    def _():
        barrier_sem = pltpu.get_barrier_semaphore()
        pl.semaphore_signal(
            barrier_sem,
            inc=1,
            device_id=nbr,
            device_id_type=pl.DeviceIdType.MESH,
        )
        pl.semaphore_wait(barrier_sem, 1)
        for j in range(C):
            _make_chunk_rdma(
                l_any_ref, lrem_ref, send_sems, recv_sems, nbr, j
            ).start()

    _make_chunk_rdma(
        l_any_ref, lrem_ref, send_sems, recv_sems, nbr, step
    ).wait_recv()
    copy = pltpu.make_async_copy(
        lrem_ref.at[pl.ds(step * BM, BM), :], lb_vmem, copy_sem
    )
    copy.start()
    copy.wait()

    a = la_ref[:, :]
    b = lb_vmem[:, :]
    m = jnp.maximum(
        jnp.max(a, axis=-1, keepdims=True), jnp.max(b, axis=-1, keepdims=True)
    )
    ea = jnp.exp(a - m)
    eb = jnp.exp(b - m)
    denom = jnp.sum(ea, axis=-1, keepdims=True) + jnp.sum(eb, axis=-1, keepdims=True)
    pa = ea / denom
    pb = eb / denom

    @pl.when(my_x == 0)
    def _():
        out_ref[:, :V_HALF] = pa
        out_ref[:, V_HALF:] = pb

    @pl.when(my_x == 1)
    def _():
        out_ref[:, :V_HALF] = pb
        out_ref[:, V_HALF:] = pa

    @pl.when(step == C - 1)
    def _():
        for j in range(C):
            _make_chunk_rdma(
                l_any_ref, lrem_ref, send_sems, recv_sems, nbr, j
            ).wait_send()


def kernel(x, W):
    l_local = jnp.dot(x, W, preferred_element_type=jnp.float32)

    out, _ = pl.pallas_call(
        _fused_body,
        grid=(C,),
        out_shape=(
            jax.ShapeDtypeStruct((T, V), jnp.float32),
            jax.ShapeDtypeStruct((T, V_HALF), jnp.float32),
        ),
        in_specs=[
            pl.BlockSpec(memory_space=pl.ANY),
            pl.BlockSpec((BM, V_HALF), lambda i: (i, 0)),
        ],
        out_specs=(
            pl.BlockSpec((BM, V), lambda i: (i, 0)),
            pl.BlockSpec(memory_space=pl.ANY),
        ),
        scratch_shapes=[
            pltpu.VMEM((BM, V_HALF), jnp.float32),
            pltpu.SemaphoreType.DMA((C,)),
            pltpu.SemaphoreType.DMA((C,)),
            pltpu.SemaphoreType.DMA,
        ],
        compiler_params=pltpu.CompilerParams(collective_id=0),
    )(l_local, l_local)

    return out


# device time: 899071 ns/iter; 1.0019x vs baseline; 1.0019x over previous
import jax
import jax.numpy as jnp
from jax import lax
from jax.experimental import pallas as pl
from jax.experimental.pallas import tpu as pltpu

T = 1024
V = 32768
V_HALF = V // 2
C = 32
BM = T // C
NSLOT = 4


def _rdma_chunk(l_any_ref, lb_slots, send_sems, recv_sems, nbr, c, slot):
    return pltpu.make_async_remote_copy(
        src_ref=l_any_ref.at[pl.ds(c * BM, BM), :],
        dst_ref=lb_slots.at[slot],
        send_sem=send_sems.at[c],
        recv_sem=recv_sems.at[slot],
        device_id=nbr,
        device_id_type=pl.DeviceIdType.MESH,
    )


def _fused_body(
    l_any_ref,
    la_ref,
    out_ref,
    lb_slots,
    send_sems,
    recv_sems,
    credit_sem,
):
    step = pl.program_id(0)
    my_x = lax.axis_index("x")
    my_y = lax.axis_index("y")
    my_z = lax.axis_index("z")
    nbr = (1 - my_x, my_y, my_z)
    slot = lax.rem(step, NSLOT)

    @pl.when(step == 0)
    def _():
        barrier_sem = pltpu.get_barrier_semaphore()
        pl.semaphore_signal(
            barrier_sem,
            inc=1,
            device_id=nbr,
            device_id_type=pl.DeviceIdType.MESH,
        )
        pl.semaphore_wait(barrier_sem, 1)
        for j in range(NSLOT):
            _rdma_chunk(
                l_any_ref, lb_slots, send_sems, recv_sems, nbr, j, j
            ).start()

    _rdma_chunk(
        l_any_ref, lb_slots, send_sems, recv_sems, nbr, step, slot
    ).wait_recv()

    a = la_ref[:, :]
    b = lb_slots[slot]
    m = jnp.maximum(
        jnp.max(a, axis=-1, keepdims=True), jnp.max(b, axis=-1, keepdims=True)
    )
    ea = jnp.exp(a - m)
    eb = jnp.exp(b - m)
    denom = jnp.sum(ea, axis=-1, keepdims=True) + jnp.sum(eb, axis=-1, keepdims=True)
    pa = ea / denom
    pb = eb / denom

    @pl.when(my_x == 0)
    def _():
        out_ref[:, :V_HALF] = pa
        out_ref[:, V_HALF:] = pb

    @pl.when(my_x == 1)
    def _():
        out_ref[:, :V_HALF] = pb
        out_ref[:, V_HALF:] = pa

    @pl.when(step < C - NSLOT)
    def _():
        pl.semaphore_signal(
            credit_sem,
            inc=1,
            device_id=nbr,
            device_id_type=pl.DeviceIdType.MESH,
        )

    @pl.when(step + NSLOT < C)
    def _():
        pl.semaphore_wait(credit_sem, 1)
        _rdma_chunk(
            l_any_ref, lb_slots, send_sems, recv_sems, nbr, step + NSLOT, slot
        ).start()

    @pl.when(step == C - 1)
    def _():
        for j in range(C):
            _rdma_chunk(
                l_any_ref, lb_slots, send_sems, recv_sems, nbr, j, j % NSLOT
            ).wait_send()


def kernel(x, W):
    l_local = jnp.dot(x, W, preferred_element_type=jnp.float32)

    return pl.pallas_call(
        _fused_body,
        grid=(C,),
        out_shape=jax.ShapeDtypeStruct((T, V), jnp.float32),
        in_specs=[
            pl.BlockSpec(memory_space=pl.ANY),
            pl.BlockSpec((BM, V_HALF), lambda i: (i, 0)),
        ],
        out_specs=pl.BlockSpec((BM, V), lambda i: (i, 0)),
        scratch_shapes=[
            pltpu.VMEM((NSLOT, BM, V_HALF), jnp.float32),
            pltpu.SemaphoreType.DMA((C,)),
            pltpu.SemaphoreType.DMA((NSLOT,)),
            pltpu.SemaphoreType.REGULAR,
        ],
        compiler_params=pltpu.CompilerParams(collective_id=0),
    )(l_local, l_local)


# device time: 823874 ns/iter; 1.0933x vs baseline; 1.0913x over previous
import jax
import jax.numpy as jnp
from jax import lax
from jax.experimental import pallas as pl
from jax.experimental.pallas import tpu as pltpu

T = 1024
D = 2048
V = 32768
V_HALF = V // 2
C = 32
BN = V_HALF // C
NSLOT = 2
BM = 32


def _rdma_chunk(slots, lrem_ref, send_sems, recv_sems, nbr, j, s):
    return pltpu.make_async_remote_copy(
        src_ref=slots.at[s],
        dst_ref=lrem_ref.at[:, pl.ds(j * BN, BN)],
        send_sem=send_sems.at[j],
        recv_sem=recv_sems.at[j],
        device_id=nbr,
        device_id_type=pl.DeviceIdType.MESH,
    )


def _local_copy(slots, ll_ref, slot_sems, j, s):
    return pltpu.make_async_copy(
        slots.at[s],
        ll_ref.at[:, pl.ds(j * BN, BN)],
        slot_sems.at[s],
    )


def _gemm_exchange_body(
    x_ref,
    w_ref,
    ll_ref,
    lrem_ref,
    slots,
    slot_sems,
    send_sems,
    recv_sems,
):
    j = pl.program_id(0)
    my_x = lax.axis_index("x")
    my_y = lax.axis_index("y")
    my_z = lax.axis_index("z")
    nbr = (1 - my_x, my_y, my_z)
    s = lax.rem(j, NSLOT)

    @pl.when(j == 0)
    def _():
        barrier_sem = pltpu.get_barrier_semaphore()
        pl.semaphore_signal(
            barrier_sem,
            inc=1,
            device_id=nbr,
            device_id_type=pl.DeviceIdType.MESH,
        )
        pl.semaphore_wait(barrier_sem, 1)

    @pl.when(j >= NSLOT)
    def _():
        _rdma_chunk(slots, lrem_ref, send_sems, recv_sems, nbr, j - NSLOT, s).wait_send()
        _local_copy(slots, ll_ref, slot_sems, j - NSLOT, s).wait()

    acc = jnp.dot(x_ref[:, :], w_ref[:, :], preferred_element_type=jnp.float32)
    slots[s] = acc
    _local_copy(slots, ll_ref, slot_sems, j, s).start()
    _rdma_chunk(slots, lrem_ref, send_sems, recv_sems, nbr, j, s).start()

    @pl.when(j == C - 1)
    def _():
        for k in range(C - NSLOT, C):
            _rdma_chunk(
                slots, lrem_ref, send_sems, recv_sems, nbr, k, k % NSLOT
            ).wait_send()
            _local_copy(slots, ll_ref, slot_sems, k, k % NSLOT).wait()
        for k in range(C):
            _rdma_chunk(
                slots, lrem_ref, send_sems, recv_sems, nbr, k, k % NSLOT
            ).wait_recv()


def _softmax_body(la_ref, lb_ref, out_ref):
    my_x = lax.axis_index("x")
    a = la_ref[:, :]
    b = lb_ref[:, :]
    m = jnp.maximum(
        jnp.max(a, axis=-1, keepdims=True), jnp.max(b, axis=-1, keepdims=True)
    )
    ea = jnp.exp(a - m)
    eb = jnp.exp(b - m)
    denom = jnp.sum(ea, axis=-1, keepdims=True) + jnp.sum(eb, axis=-1, keepdims=True)
    pa = ea / denom
    pb = eb / denom

    @pl.when(my_x == 0)
    def _():
        out_ref[:, :V_HALF] = pa
        out_ref[:, V_HALF:] = pb

    @pl.when(my_x == 1)
    def _():
        out_ref[:, :V_HALF] = pb
        out_ref[:, V_HALF:] = pa


def kernel(x, W):
    l_local, l_remote = pl.pallas_call(
        _gemm_exchange_body,
        grid=(C,),
        out_shape=(
            jax.ShapeDtypeStruct((T, V_HALF), jnp.float32),
            jax.ShapeDtypeStruct((T, V_HALF), jnp.float32),
        ),
        in_specs=[
            pl.BlockSpec((T, D), lambda j: (0, 0)),
            pl.BlockSpec((D, BN), lambda j: (0, j)),
        ],
        out_specs=(
            pl.BlockSpec(memory_space=pl.ANY),
            pl.BlockSpec(memory_space=pl.ANY),
        ),
        scratch_shapes=[
            pltpu.VMEM((NSLOT, T, BN), jnp.float32),
            pltpu.SemaphoreType.DMA((NSLOT,)),
            pltpu.SemaphoreType.DMA((C,)),
            pltpu.SemaphoreType.DMA((C,)),
        ],
        compiler_params=pltpu.CompilerParams(collective_id=0),
    )(x, W)

    return pl.pallas_call(
        _softmax_body,
        grid=(T // BM,),
        out_shape=jax.ShapeDtypeStruct((T, V), jnp.float32),
        in_specs=[
            pl.BlockSpec((BM, V_HALF), lambda i: (i, 0)),
            pl.BlockSpec((BM, V_HALF), lambda i: (i, 0)),
        ],
        out_specs=pl.BlockSpec((BM, V), lambda i: (i, 0)),
    )(l_local, l_remote)


# device time: 820345 ns/iter; 1.0980x vs baseline; 1.0043x over previous
import jax
import jax.numpy as jnp
from jax import lax
from jax.experimental import pallas as pl
from jax.experimental.pallas import tpu as pltpu

T = 1024
D = 2048
V = 32768
V_HALF = V // 2
C = 32
BN = V_HALF // C
NSLOT = 2
BM = 64


def _rdma_chunk(slots, lrem_ref, send_sems, recv_sems, nbr, j, s):
    return pltpu.make_async_remote_copy(
        src_ref=slots.at[s],
        dst_ref=lrem_ref.at[:, pl.ds(j * BN, BN)],
        send_sem=send_sems.at[j],
        recv_sem=recv_sems.at[j],
        device_id=nbr,
        device_id_type=pl.DeviceIdType.MESH,
    )


def _local_copy(slots, ll_ref, slot_sems, j, s):
    return pltpu.make_async_copy(
        slots.at[s],
        ll_ref.at[:, pl.ds(j * BN, BN)],
        slot_sems.at[s],
    )


def _gemm_exchange_body(
    x_ref,
    w_ref,
    ll_ref,
    lrem_ref,
    slots,
    slot_sems,
    send_sems,
    recv_sems,
):
    j = pl.program_id(0)
    my_x = lax.axis_index("x")
    my_y = lax.axis_index("y")
    my_z = lax.axis_index("z")
    nbr = (1 - my_x, my_y, my_z)
    s = lax.rem(j, NSLOT)

    @pl.when(j == 0)
    def _():
        barrier_sem = pltpu.get_barrier_semaphore()
        pl.semaphore_signal(
            barrier_sem,
            inc=1,
            device_id=nbr,
            device_id_type=pl.DeviceIdType.MESH,
        )
        pl.semaphore_wait(barrier_sem, 1)

    @pl.when(j >= NSLOT)
    def _():
        _rdma_chunk(slots, lrem_ref, send_sems, recv_sems, nbr, j - NSLOT, s).wait_send()
        _local_copy(slots, ll_ref, slot_sems, j - NSLOT, s).wait()

    acc = jnp.dot(x_ref[:, :], w_ref[:, :], preferred_element_type=jnp.float32)
    slots[s] = acc
    _local_copy(slots, ll_ref, slot_sems, j, s).start()
    _rdma_chunk(slots, lrem_ref, send_sems, recv_sems, nbr, j, s).start()

    @pl.when(j == C - 1)
    def _():
        for k in range(C - NSLOT, C):
            _rdma_chunk(
                slots, lrem_ref, send_sems, recv_sems, nbr, k, k % NSLOT
            ).wait_send()
            _local_copy(slots, ll_ref, slot_sems, k, k % NSLOT).wait()
        for k in range(C):
            _rdma_chunk(
                slots, lrem_ref, send_sems, recv_sems, nbr, k, k % NSLOT
            ).wait_recv()


def _softmax_body(la_ref, lb_ref, out_ref):
    my_x = lax.axis_index("x")
    a = la_ref[:, :]
    b = lb_ref[:, :]
    m = jnp.maximum(
        jnp.max(a, axis=-1, keepdims=True), jnp.max(b, axis=-1, keepdims=True)
    )
    ea = jnp.exp(a - m)
    eb = jnp.exp(b - m)
    denom = jnp.sum(ea, axis=-1, keepdims=True) + jnp.sum(eb, axis=-1, keepdims=True)
    inv = 1.0 / denom
    pa = ea * inv
    pb = eb * inv

    @pl.when(my_x == 0)
    def _():
        out_ref[:, :V_HALF] = pa
        out_ref[:, V_HALF:] = pb

    @pl.when(my_x == 1)
    def _():
        out_ref[:, :V_HALF] = pb
        out_ref[:, V_HALF:] = pa


def kernel(x, W):
    l_local, l_remote = pl.pallas_call(
        _gemm_exchange_body,
        grid=(C,),
        out_shape=(
            jax.ShapeDtypeStruct((T, V_HALF), jnp.float32),
            jax.ShapeDtypeStruct((T, V_HALF), jnp.float32),
        ),
        in_specs=[
            pl.BlockSpec((T, D), lambda j: (0, 0)),
            pl.BlockSpec((D, BN), lambda j: (0, j)),
        ],
        out_specs=(
            pl.BlockSpec(memory_space=pl.ANY),
            pl.BlockSpec(memory_space=pl.ANY),
        ),
        scratch_shapes=[
            pltpu.VMEM((NSLOT, T, BN), jnp.float32),
            pltpu.SemaphoreType.DMA((NSLOT,)),
            pltpu.SemaphoreType.DMA((C,)),
            pltpu.SemaphoreType.DMA((C,)),
        ],
        compiler_params=pltpu.CompilerParams(collective_id=0),
    )(x, W)

    return pl.pallas_call(
        _softmax_body,
        grid=(T // BM,),
        out_shape=jax.ShapeDtypeStruct((T, V), jnp.float32),
        in_specs=[
            pl.BlockSpec((BM, V_HALF), lambda i: (i, 0)),
            pl.BlockSpec((BM, V_HALF), lambda i: (i, 0)),
        ],
        out_specs=pl.BlockSpec((BM, V), lambda i: (i, 0)),
        compiler_params=pltpu.CompilerParams(
            vmem_limit_bytes=56 * 1024 * 1024,
        ),
    )(l_local, l_remote)
